# baseline (speedup 1.0000x reference)
import functools

import jax
import jax.numpy as jnp
from jax import lax
from jax.experimental import pallas as pl
from jax.experimental.pallas import tpu as pltpu
from jax.experimental.pallas import tpu_sc as plsc

BATCH = 4096
SEQ = 77
DIM = 768
LANES = 16
NCORES = 2
NSUB = 16
NW = NCORES * NSUB

CHUNK = 32
NB = BATCH // CHUNK
NTILES = SEQ * NB
TPW = NTILES // NW
NBUF = 4
SEQ_PAD = 80
NPOS = 16


def _sc_embed(idx_flat, token_table, pos_table):
    mesh = plsc.VectorSubcoreMesh(core_axis_name="core", subcore_axis_name="sub")

    scratch = [
        pltpu.VMEM((TPW * CHUNK,), jnp.int32),
    ]
    for _ in range(NBUF):
        scratch += [
            pltpu.VMEM((CHUNK,), jnp.int32),
            pltpu.VMEM((CHUNK, DIM), jnp.float32),
            pltpu.VMEM((DIM,), jnp.float32),
            pltpu.SemaphoreType.DMA,
            pltpu.SemaphoreType.DMA,
        ]

    @functools.partial(
        pl.kernel,
        mesh=mesh,
        out_type=jax.ShapeDtypeStruct((SEQ, BATCH, DIM), jnp.float32),
        scratch_types=scratch,
    )
    def k(idx_hbm, table_hbm, pos_hbm, out_hbm, idx_all, *bufs):
        wid = lax.axis_index("sub") * NCORES + lax.axis_index("core")
        t0 = wid * TPW
        rings = [tuple(bufs[5 * b:5 * b + 5]) for b in range(NBUF)]

        pltpu.sync_copy(idx_hbm.at[pl.ds(t0 * CHUNK, TPW * CHUNK)], idx_all)

        def coords(kk):
            t = t0 + kk
            s_i = t // NB
            b0 = (t % NB) * CHUNK
            return s_i, b0

        def prep_inputs(kk, idx_v, pos_v):
            s_i, _ = coords(kk)
            pltpu.sync_copy(pos_hbm.at[s_i], pos_v)
            for q in range(CHUNK // LANES):
                idx_v[pl.ds(q * LANES, LANES)] = (
                    idx_all[pl.ds(kk * CHUNK + q * LANES, LANES)])

        def start_gather(kk, idx_v, rows_v, pos_v, gsem):
            prep_inputs(kk, idx_v, pos_v)
            pltpu.async_copy(table_hbm.at[idx_v], rows_v, gsem)

        def wait_gather(idx_v, rows_v, gsem):
            pltpu.make_async_copy(table_hbm.at[idx_v], rows_v, gsem).wait()

        def wait_store(kk, rows_v, ssem):
            s_i, b0 = coords(kk)
            pltpu.make_async_copy(
                rows_v, out_hbm.at[s_i, pl.ds(b0, CHUNK)], ssem).wait()

        for b in range(NBUF):
            idx_v, rows_v, pos_v, gsem, _ = rings[b]
            start_gather(b, idx_v, rows_v, pos_v, gsem)

        def outer(jj, carry):
            for b in range(NBUF):
                idx_v, rows_v, pos_v, gsem, ssem = rings[b]
                kk = jj * NBUF + b
                s_i, b0 = coords(kk)
                wait_gather(idx_v, rows_v, gsem)
                DDB = 12
                for blk in range(DIM // LANES // DDB):
                    pvecs = [pos_v[pl.ds((blk * DDB + q) * LANES, LANES)]
                             for q in range(DDB)]

                    def row_add(r, c):
                        for q in range(DDB):
                            sl = pl.ds((blk * DDB + q) * LANES, LANES)
                            rows_v[r, sl] = rows_v[r, sl] + pvecs[q]
                        return c

                    lax.fori_loop(0, CHUNK, row_add, 0)
                pltpu.async_copy(
                    rows_v, out_hbm.at[s_i, pl.ds(b0, CHUNK)], ssem)

                @pl.when(kk + NBUF < TPW)
                def _prep_next():
                    prep_inputs(kk + NBUF, idx_v, pos_v)
                    wait_store(kk, rows_v, ssem)
                    pltpu.async_copy(table_hbm.at[idx_v], rows_v, gsem)
            return carry

        lax.fori_loop(0, TPW // NBUF, outer, 0)

        for b in range(NBUF):
            _, rows_v, _, _, ssem = rings[b]
            wait_store(TPW - NBUF + b, rows_v, ssem)

    return k(idx_flat, token_table, pos_table)


def kernel(input_tokens, token_table, pos_table):
    idx_flat = input_tokens.astype(jnp.int32).T.reshape(-1)
    out = _sc_embed(idx_flat, token_table, pos_table)
    return jnp.transpose(out, (1, 0, 2))


# speedup vs baseline: 1.0964x; 1.0964x over previous
import functools

import jax
import jax.numpy as jnp
from jax import lax
from jax.experimental import pallas as pl
from jax.experimental.pallas import tpu as pltpu
from jax.experimental.pallas import tpu_sc as plsc

BATCH = 4096
SEQ = 77
DIM = 768
LANES = 16
NCORES = 2
NSUB = 16
NW = NCORES * NSUB

CHUNK = 64
NB = BATCH // CHUNK
NTILES = SEQ * NB
TPW = NTILES // NW
NBUF = 2
NQ = 4
QROWS = CHUNK // NQ
SEQ_PAD = 80
NPOS = 16


def _sc_embed(idx_flat, token_table, pos_table):
    mesh = plsc.VectorSubcoreMesh(core_axis_name="core", subcore_axis_name="sub")

    scratch = [
        pltpu.VMEM((TPW * CHUNK,), jnp.int32),
    ]
    for _ in range(NBUF):
        scratch += [
            pltpu.VMEM((CHUNK,), jnp.int32),
            pltpu.VMEM((CHUNK, DIM), jnp.float32),
            pltpu.VMEM((DIM,), jnp.float32),
            pltpu.SemaphoreType.DMA,
            pltpu.SemaphoreType.DMA,
        ]

    @functools.partial(
        pl.kernel,
        mesh=mesh,
        out_type=jax.ShapeDtypeStruct((SEQ, BATCH, DIM), jnp.float32),
        scratch_types=scratch,
    )
    def k(idx_hbm, table_hbm, pos_hbm, out_hbm, idx_all, *bufs):
        wid = lax.axis_index("sub") * NCORES + lax.axis_index("core")
        t0 = wid * TPW
        rings = [tuple(bufs[5 * b:5 * b + 5]) for b in range(NBUF)]

        pltpu.sync_copy(idx_hbm.at[pl.ds(t0 * CHUNK, TPW * CHUNK)], idx_all)

        def coords(kk):
            t = t0 + kk
            s_i = t // NB
            b0 = (t % NB) * CHUNK
            return s_i, b0

        def prep_inputs(kk, idx_v, pos_v):
            s_i, _ = coords(kk)
            pltpu.sync_copy(pos_hbm.at[s_i], pos_v)
            for q in range(CHUNK // LANES):
                idx_v[pl.ds(q * LANES, LANES)] = (
                    idx_all[pl.ds(kk * CHUNK + q * LANES, LANES)])

        def start_gather(kk, idx_v, rows_v, pos_v, gsem):
            prep_inputs(kk, idx_v, pos_v)
            pltpu.async_copy(table_hbm.at[idx_v], rows_v, gsem)

        def wait_gather(idx_v, rows_v, gsem):
            pltpu.make_async_copy(table_hbm.at[idx_v], rows_v, gsem).wait()

        def wait_store(kk, rows_v, ssem):
            s_i, b0 = coords(kk)
            for qt in range(NQ):
                pltpu.make_async_copy(
                    rows_v.at[pl.ds(qt * QROWS, QROWS)],
                    out_hbm.at[s_i, pl.ds(b0 + qt * QROWS, QROWS)],
                    ssem).wait()

        for b in range(NBUF):
            idx_v, rows_v, pos_v, gsem, _ = rings[b]
            start_gather(b, idx_v, rows_v, pos_v, gsem)

        def outer(jj, carry):
            for b in range(NBUF):
                idx_v, rows_v, pos_v, gsem, ssem = rings[b]
                kk = jj * NBUF + b
                s_i, b0 = coords(kk)
                wait_gather(idx_v, rows_v, gsem)
                pvecs = [pos_v[pl.ds(dd * LANES, LANES)]
                         for dd in range(DIM // LANES)]

                def row_add(r, c):
                    for dd in range(DIM // LANES):
                        sl = pl.ds(dd * LANES, LANES)
                        rows_v[r, sl] = rows_v[r, sl] + pvecs[dd]
                    return c

                for qt in range(NQ):
                    lax.fori_loop(qt * QROWS, (qt + 1) * QROWS, row_add, 0)
                    pltpu.async_copy(
                        rows_v.at[pl.ds(qt * QROWS, QROWS)],
                        out_hbm.at[s_i, pl.ds(b0 + qt * QROWS, QROWS)],
                        ssem)

                @pl.when(kk + NBUF < TPW)
                def _prep_next():
                    prep_inputs(kk + NBUF, idx_v, pos_v)
                    wait_store(kk, rows_v, ssem)
                    pltpu.async_copy(table_hbm.at[idx_v], rows_v, gsem)
            return carry

        lax.fori_loop(0, TPW // NBUF, outer, 0)

        for b in range(NBUF):
            _, rows_v, _, _, ssem = rings[b]
            wait_store(TPW - NBUF + b, rows_v, ssem)

    return k(idx_flat, token_table, pos_table)


def kernel(input_tokens, token_table, pos_table):
    idx_flat = input_tokens.astype(jnp.int32).T.reshape(-1)
    out = _sc_embed(idx_flat, token_table, pos_table)
    return jnp.transpose(out, (1, 0, 2))
